# baseline (speedup 1.0000x reference)
import functools

import jax
import jax.numpy as jnp
from jax import lax
from jax.experimental import pallas as pl
from jax.experimental.pallas import tpu as pltpu
import jax.experimental.pallas.tpu_sc as plsc

_K = 32
_NC, _NS = 2, 16
_NW = _NC * _NS
_L = 16

_P0, _P1, _P2 = 10240, 2560, 768


def _mesh():
    return plsc.VectorSubcoreMesh(core_axis_name="c", subcore_axis_name="s",
                                  num_cores=_NC, num_subcores=_NS)


def _wid():
    return lax.axis_index("s") * _NC + lax.axis_index("c")


def _sc_gather_sum(table2, idx3, P, CB, D):
    dc = D // 128
    R = CB * dc
    G = P // CB
    nch = G // _NW

    @functools.partial(
        pl.kernel,
        out_type=jax.ShapeDtypeStruct((P * dc, 128), jnp.float32),
        mesh=_mesh(),
        scratch_types=[
            pltpu.VMEM((nch, _K, R), jnp.int32),
            pltpu.VMEM((R, 128), jnp.float32),
            pltpu.VMEM((R, 128), jnp.float32),
            pltpu.SemaphoreType.DMA,
            pltpu.SemaphoreType.DMA,
        ],
    )
    def k(table_hbm, idx_hbm, out_hbm, idx_v, acc0, acc1, sem0, sem1):
        w = _wid()
        z = jnp.zeros((_L,), jnp.float32)
        pltpu.sync_copy(idx_hbm.at[w], idx_v)

        def zero(acc):
            def zrow(i, carry):
                for d in range(128 // _L):
                    acc[i, pl.ds(d * _L, _L)] = z
                return carry

            lax.fori_loop(0, R, zrow, 0)

        def fire(c, acc, sem):
            for kk in range(_K):
                pltpu.async_copy(table_hbm.at[idx_v.at[c, kk]], acc, sem,
                                 add=True)

        def drain_wb(c, acc, sem):
            for kk in range(_K):
                pltpu.make_async_copy(table_hbm.at[pl.ds(0, R)], acc,
                                      sem).wait()
            pltpu.sync_copy(acc, out_hbm.at[pl.ds((w * nch + c) * R, R)])

        zero(acc0)
        fire(0, acc0, sem0)

        def body(c, carry):
            @pl.when(c % 2 == 1)
            def _():
                zero(acc1)
                fire(c, acc1, sem1)
                drain_wb(c - 1, acc0, sem0)

            @pl.when(c % 2 == 0)
            def _():
                zero(acc0)
                fire(c, acc0, sem0)
                drain_wb(c - 1, acc1, sem1)

            return carry

        lax.fori_loop(1, nch, body, 0)
        if nch % 2 == 1:
            drain_wb(nch - 1, acc0, sem0)
        else:
            drain_wb(nch - 1, acc1, sem1)

    return k(table2, idx3).reshape(P, D)


def _sc_gather_max(table, idx2, P, CB, D):
    G = P // CB
    nch = G // _NW
    M = CB * _K

    @functools.partial(
        pl.kernel,
        out_type=jax.ShapeDtypeStruct((P, D), jnp.float32),
        mesh=_mesh(),
        scratch_types=[
            pltpu.VMEM((nch, M), jnp.int32),
            pltpu.VMEM((M, D), jnp.float32),
            pltpu.VMEM((M, D), jnp.float32),
            pltpu.VMEM((CB, D), jnp.float32),
            pltpu.SemaphoreType.DMA,
            pltpu.SemaphoreType.DMA,
        ],
    )
    def k(table_hbm, idx_hbm, out_hbm, idx_v, rows0, rows1, out_v, sem0, sem1):
        w = _wid()
        pltpu.sync_copy(idx_hbm.at[w], idx_v)

        def compute_wb(c, rows_v):
            for p in range(CB):
                init = tuple(
                    rows_v[p * _K, pl.ds(d * _L, _L)] for d in range(D // _L)
                )

                def kbody(kk, m):
                    return tuple(
                        jnp.maximum(m[d],
                                    rows_v[p * _K + kk, pl.ds(d * _L, _L)])
                        for d in range(D // _L)
                    )

                m = lax.fori_loop(1, _K, kbody, init)
                for d in range(D // _L):
                    out_v[p, pl.ds(d * _L, _L)] = m[d]
            pltpu.sync_copy(out_v, out_hbm.at[pl.ds((w * nch + c) * CB, CB)])

        pltpu.async_copy(table_hbm.at[idx_v.at[0]], rows0, sem0)

        def body(c, carry):
            @pl.when(c % 2 == 1)
            def _():
                pltpu.async_copy(table_hbm.at[idx_v.at[c]], rows1, sem1)
                pltpu.make_async_copy(table_hbm.at[pl.ds(0, M)], rows0,
                                      sem0).wait()
                compute_wb(c - 1, rows0)

            @pl.when(c % 2 == 0)
            def _():
                pltpu.async_copy(table_hbm.at[idx_v.at[c]], rows0, sem0)
                pltpu.make_async_copy(table_hbm.at[pl.ds(0, M)], rows1,
                                      sem1).wait()
                compute_wb(c - 1, rows1)

            return carry

        lax.fori_loop(1, nch, body, 0)
        if nch % 2 == 1:
            pltpu.make_async_copy(table_hbm.at[pl.ds(0, M)], rows0, sem0).wait()
            compute_wb(nch - 1, rows0)
        else:
            pltpu.make_async_copy(table_hbm.at[pl.ds(0, M)], rows1, sem1).wait()
            compute_wb(nch - 1, rows1)

    return k(table, idx2)


def _sc_gather_rows(table, idx2, P, CB, D):
    G = P // CB
    nch = G // _NW

    @functools.partial(
        pl.kernel,
        out_type=jax.ShapeDtypeStruct((P, D), jnp.float32),
        mesh=_mesh(),
        scratch_types=[
            pltpu.VMEM((nch, CB), jnp.int32),
            pltpu.VMEM((CB, D), jnp.float32),
            pltpu.VMEM((CB, D), jnp.float32),
            pltpu.SemaphoreType.DMA,
            pltpu.SemaphoreType.DMA,
        ],
    )
    def k(table_hbm, idx_hbm, out_hbm, idx_v, rows0, rows1, sem0, sem1):
        w = _wid()
        pltpu.sync_copy(idx_hbm.at[w], idx_v)
        pltpu.async_copy(table_hbm.at[idx_v.at[0]], rows0, sem0)

        def wb(c, rows, sem):
            pltpu.make_async_copy(table_hbm.at[pl.ds(0, CB)], rows, sem).wait()
            pltpu.sync_copy(rows, out_hbm.at[pl.ds((w * nch + c) * CB, CB)])

        def body(c, carry):
            @pl.when(c % 2 == 1)
            def _():
                pltpu.async_copy(table_hbm.at[idx_v.at[c]], rows1, sem1)
                wb(c - 1, rows0, sem0)

            @pl.when(c % 2 == 0)
            def _():
                pltpu.async_copy(table_hbm.at[idx_v.at[c]], rows0, sem0)
                wb(c - 1, rows1, sem1)

            return carry

        lax.fori_loop(1, nch, body, 0)
        if nch % 2 == 1:
            wb(nch - 1, rows0, sem0)
        else:
            wb(nch - 1, rows1, sem1)

    return k(table, idx2)


def _tc_mm(x, W, scale=None, relu=True, br=512):
    N, Di = x.shape
    Do = W.shape[1]

    def body(x_ref, w_ref, o_ref):
        xb = x_ref[...]
        if scale is not None:
            xb = xb * scale
        y = jnp.dot(xb, w_ref[...], preferred_element_type=jnp.float32)
        if relu:
            y = jnp.maximum(y, 0.0)
        o_ref[...] = y

    return pl.pallas_call(
        body,
        grid=(N // br,),
        in_specs=[
            pl.BlockSpec((br, Di), lambda i: (i, 0)),
            pl.BlockSpec((Di, Do), lambda i: (0, 0)),
        ],
        out_specs=pl.BlockSpec((br, Do), lambda i: (i, 0)),
        out_shape=jax.ShapeDtypeStruct((N, Do), jnp.float32),
    )(x, W)


def _tc_enc2_lat(s2, W_enc2, Wl1b):
    N, D = s2.shape
    Do = Wl1b.shape[1]

    def body(s_ref, we_ref, wb_ref, x2_ref, z2_ref):
        x2 = jnp.maximum(
            jnp.dot(s_ref[...] * (1.0 / _K), we_ref[...],
                    preferred_element_type=jnp.float32), 0.0)
        x2_ref[...] = x2
        z2_ref[...] = jnp.dot(x2, wb_ref[...], preferred_element_type=jnp.float32)

    return pl.pallas_call(
        body,
        out_shape=(
            jax.ShapeDtypeStruct((N, D), jnp.float32),
            jax.ShapeDtypeStruct((N, Do), jnp.float32),
        ),
    )(s2, W_enc2, Wl1b)


def _tc_lat1(x1, u1, Wl1a, Wlb, br=512):
    N, D = x1.shape
    Do = Wlb.shape[1]

    def body(x_ref, u_ref, wa_ref, wb_ref, o_ref):
        h = jnp.maximum(
            jnp.dot(x_ref[...], wa_ref[...], preferred_element_type=jnp.float32)
            + u_ref[...], 0.0)
        o_ref[...] = jnp.dot(h, wb_ref[...], preferred_element_type=jnp.float32)

    return pl.pallas_call(
        body,
        grid=(N // br,),
        in_specs=[
            pl.BlockSpec((br, D), lambda i: (i, 0)),
            pl.BlockSpec((br, D), lambda i: (i, 0)),
            pl.BlockSpec((D, D), lambda i: (0, 0)),
            pl.BlockSpec((D, Do), lambda i: (0, 0)),
        ],
        out_specs=pl.BlockSpec((br, Do), lambda i: (i, 0)),
        out_shape=jax.ShapeDtypeStruct((N, Do), jnp.float32),
    )(x1, u1, Wl1a, Wlb)


def _tc_head(x0, u0, Wla, W_head, W_out, br=1024):
    N, D = x0.shape
    C = W_out.shape[1]

    def body(x_ref, u_ref, wa_ref, wh_ref, wo_ref, o_ref):
        t = jnp.maximum(
            jnp.dot(x_ref[...], wa_ref[...], preferred_element_type=jnp.float32)
            + u_ref[...], 0.0)
        t = jnp.maximum(
            jnp.dot(t, wh_ref[...], preferred_element_type=jnp.float32), 0.0)
        o_ref[...] = jnp.dot(t, wo_ref[...], preferred_element_type=jnp.float32)

    return pl.pallas_call(
        body,
        grid=(N // br,),
        in_specs=[
            pl.BlockSpec((br, D), lambda i: (i, 0)),
            pl.BlockSpec((br, D), lambda i: (i, 0)),
            pl.BlockSpec((D, D), lambda i: (0, 0)),
            pl.BlockSpec((D, D), lambda i: (0, 0)),
            pl.BlockSpec((D, C), lambda i: (0, 0)),
        ],
        out_specs=pl.BlockSpec((br, C), lambda i: (i, 0)),
        out_shape=jax.ShapeDtypeStruct((N, C), jnp.float32),
    )(x0, u0, Wla, W_head, W_out)


def _pack_neigh(n, P, CB, dc):
    G = P // CB
    n = jnp.pad(n, ((0, P - n.shape[0]), (0, 0))).astype(jnp.int32)
    q = n.reshape(G, CB, _K) * dc
    q = q[:, :, :, None] + jnp.arange(dc, dtype=jnp.int32)
    q = q.transpose(0, 2, 1, 3).reshape(G, _K, CB * dc)
    return q.reshape(_NW, G // _NW, _K, CB * dc)


def _pack_pool(p, P, CB):
    G = P // CB
    p = jnp.pad(p, ((0, P - p.shape[0]), (0, 0)))
    return p.reshape(_NW, G // _NW, CB * _K).astype(jnp.int32)


def _pack_ups(u, P, CB):
    G = P // CB
    u = jnp.pad(u, (0, P - u.shape[0]))
    return u.reshape(_NW, G // _NW, CB).astype(jnp.int32)


def kernel(features, neighbors0, neighbors1, neighbors2, pools1, pools2,
           upsamples0, upsamples1,
           W_enc0, W_pool1, W_enc1, W_pool2, W_enc2, W_lat1, W_last, W_head,
           W_out):
    D0, D1, D2 = 128, 256, 512
    N0 = features.shape[0]

    n0 = _pack_neigh(neighbors0, _P0, 32, 1)
    n1 = _pack_neigh(neighbors1, _P1, 20, 2)
    n2 = _pack_neigh(neighbors2, _P2, 12, 4)
    p1 = _pack_pool(pools1, _P1, 4)
    p2 = _pack_pool(pools2, _P2, 4)
    u0 = _pack_ups(upsamples0, _P0, 64)
    u1 = _pack_ups(upsamples1, _P1, 80)

    Wl1a, Wl1b = W_lat1[:D1], W_lat1[D1:]
    Wla, Wlb = W_last[:D0], W_last[D0:]

    s0 = _sc_gather_sum(features, n0, _P0, 32, D0)
    x0 = _tc_mm(s0, W_enc0, scale=1.0 / _K, br=1024)
    m1 = _sc_gather_max(x0, p1, _P1, 4, D0)
    h1 = _tc_mm(m1, W_pool1, br=512)
    s1 = _sc_gather_sum(h1.reshape(_P1 * 2, 128), n1, _P1, 20, D1)
    x1 = _tc_mm(s1, W_enc1, scale=1.0 / _K, br=512)
    m2 = _sc_gather_max(x1, p2, _P2, 4, D1)
    h2 = _tc_mm(m2, W_pool2, br=768)
    s2 = _sc_gather_sum(h2.reshape(_P2 * 4, 128), n2, _P2, 12, D2)
    x2, z2 = _tc_enc2_lat(s2, W_enc2, Wl1b)

    uu1 = _sc_gather_rows(z2, u1, _P1, 80, D1)
    z1 = _tc_lat1(x1, uu1, Wl1a, Wlb, br=512)
    uu0 = _sc_gather_rows(z1, u0, _P0, 64, D0)
    logits = _tc_head(x0, uu0, Wla, W_head, W_out, br=1024)

    return logits[:N0]


# speedup vs baseline: 4.5377x; 4.5377x over previous
import functools

import jax
import jax.numpy as jnp
from jax import lax
from jax.experimental import pallas as pl
from jax.experimental.pallas import tpu as pltpu
import jax.experimental.pallas.tpu_sc as plsc

_K = 32
_NC, _NS = 2, 16
_NW = _NC * _NS
_L = 16

_P0, _P1, _P2 = 10240, 2560, 768


def _mesh():
    return plsc.VectorSubcoreMesh(core_axis_name="c", subcore_axis_name="s",
                                  num_cores=_NC, num_subcores=_NS)


def _wid():
    return lax.axis_index("s") * _NC + lax.axis_index("c")


def _stage_tables(tables, sh, V):
    @pl.when(lax.axis_index("s") == 0)
    def _():
        for cc, t in enumerate(tables):
            pltpu.sync_copy(t, sh.at[pl.ds(cc * V, V)])


def _zero_rows(buf, n):
    z = jnp.zeros((_L,), jnp.float32)

    def zrow(i, carry):
        for dd in range(128 // _L):
            buf[i, pl.ds(dd * _L, _L)] = z
        return carry

    lax.fori_loop(0, n, zrow, 0)


def _sc_gather_sum(tables, idx3, P, CB):
    dc = len(tables)
    R = CB * dc
    G = P // CB
    nch = G // _NW
    V = tables[0].shape[0]

    @functools.partial(
        pl.kernel,
        out_type=tuple(
            jax.ShapeDtypeStruct((P, 128), jnp.float32) for _ in range(dc)),
        mesh=_mesh(),
        scratch_types=[
            pltpu.VMEM((nch, _K, R), jnp.int32),
            pltpu.VMEM((R, 128), jnp.float32),
            pltpu.VMEM((R, 128), jnp.float32),
            pltpu.VMEM_SHARED((dc * V, 128), jnp.float32),
            pltpu.SemaphoreType.DMA,
            pltpu.SemaphoreType.DMA,
        ],
    )
    def k(*refs):
        tabs = refs[:dc]
        idx_hbm = refs[dc]
        outs = refs[dc + 1:dc + 1 + dc]
        idx_v, acc0, acc1, sh, sem0, sem1 = refs[dc + 1 + dc:]
        w = _wid()
        _stage_tables(tabs, sh, V)
        pltpu.sync_copy(idx_hbm.at[w], idx_v)
        plsc.subcore_barrier()

        def fire(c, acc, sem):
            for kk in range(_K):
                pltpu.async_copy(sh.at[idx_v.at[c, kk]], acc, sem, add=True)

        def drain_wb(c, acc, sem):
            for kk in range(_K):
                pltpu.make_async_copy(outs[0].at[pl.ds(0, R)], acc,
                                      sem).wait()
            for cc in range(dc):
                pltpu.sync_copy(
                    acc.at[pl.ds(cc * CB, CB)],
                    outs[cc].at[pl.ds((w * nch + c) * CB, CB)])

        _zero_rows(acc0, R)
        fire(0, acc0, sem0)

        def body(c, carry):
            @pl.when(c % 2 == 1)
            def _():
                _zero_rows(acc1, R)
                fire(c, acc1, sem1)
                drain_wb(c - 1, acc0, sem0)

            @pl.when(c % 2 == 0)
            def _():
                _zero_rows(acc0, R)
                fire(c, acc0, sem0)
                drain_wb(c - 1, acc1, sem1)

            return carry

        lax.fori_loop(1, nch, body, 0)
        if nch % 2 == 1:
            drain_wb(nch - 1, acc0, sem0)
        else:
            drain_wb(nch - 1, acc1, sem1)

    return k(*tables, idx3)


def _sc_gather_max(tables, idx2, P, CB):
    dc = len(tables)
    G = P // CB
    nch = G // _NW
    M = CB * _K
    R = M * dc
    V = tables[0].shape[0]

    @functools.partial(
        pl.kernel,
        out_type=tuple(
            jax.ShapeDtypeStruct((P, 128), jnp.float32) for _ in range(dc)),
        mesh=_mesh(),
        scratch_types=[
            pltpu.VMEM((nch, R), jnp.int32),
            pltpu.VMEM((R, 128), jnp.float32),
            pltpu.VMEM((R, 128), jnp.float32),
            pltpu.VMEM((dc, CB, 128), jnp.float32),
            pltpu.VMEM_SHARED((dc * V, 128), jnp.float32),
            pltpu.SemaphoreType.DMA,
            pltpu.SemaphoreType.DMA,
        ],
    )
    def k(*refs):
        tabs = refs[:dc]
        idx_hbm = refs[dc]
        outs = refs[dc + 1:dc + 1 + dc]
        idx_v, rows0, rows1, out_v, sh, sem0, sem1 = refs[dc + 1 + dc:]
        w = _wid()
        _stage_tables(tabs, sh, V)
        pltpu.sync_copy(idx_hbm.at[w], idx_v)
        plsc.subcore_barrier()

        def fire(c, rows, sem):
            pltpu.async_copy(sh.at[idx_v.at[c]], rows, sem, add=True)

        def compute_wb(c, rows_v):
            for p in range(CB):
                for cc in range(dc):
                    base = (cc * CB + p) * _K
                    init = tuple(
                        rows_v[base, pl.ds(dd * _L, _L)]
                        for dd in range(128 // _L)
                    )

                    def kbody(kk, m):
                        return tuple(
                            jnp.maximum(m[dd],
                                        rows_v[base + kk, pl.ds(dd * _L, _L)])
                            for dd in range(128 // _L)
                        )

                    m = lax.fori_loop(1, _K, kbody, init)
                    for dd in range(128 // _L):
                        out_v[cc, p, pl.ds(dd * _L, _L)] = m[dd]
            for cc in range(dc):
                pltpu.sync_copy(
                    out_v.at[cc],
                    outs[cc].at[pl.ds((w * nch + c) * CB, CB)])

        _zero_rows(rows0, R)
        fire(0, rows0, sem0)
        _zero_rows(rows1, R)

        def body(c, carry):
            @pl.when(c % 2 == 1)
            def _():
                fire(c, rows1, sem1)
                pltpu.make_async_copy(outs[0].at[pl.ds(0, R)], rows0,
                                      sem0).wait()
                compute_wb(c - 1, rows0)
                _zero_rows(rows0, R)

            @pl.when(c % 2 == 0)
            def _():
                fire(c, rows0, sem0)
                pltpu.make_async_copy(outs[0].at[pl.ds(0, R)], rows1,
                                      sem1).wait()
                compute_wb(c - 1, rows1)
                _zero_rows(rows1, R)

            return carry

        lax.fori_loop(1, nch, body, 0)
        if nch % 2 == 1:
            pltpu.make_async_copy(outs[0].at[pl.ds(0, R)], rows0,
                                      sem0).wait()
            compute_wb(nch - 1, rows0)
        else:
            pltpu.make_async_copy(outs[0].at[pl.ds(0, R)], rows1,
                                      sem1).wait()
            compute_wb(nch - 1, rows1)

    return k(*tables, idx2)


def _sc_gather_rows(tables, idx2, P, CB):
    dc = len(tables)
    R = CB * dc
    G = P // CB
    nch = G // _NW
    V = tables[0].shape[0]

    @functools.partial(
        pl.kernel,
        out_type=tuple(
            jax.ShapeDtypeStruct((P, 128), jnp.float32) for _ in range(dc)),
        mesh=_mesh(),
        scratch_types=[
            pltpu.VMEM((nch, R), jnp.int32),
            pltpu.VMEM((R, 128), jnp.float32),
            pltpu.VMEM((R, 128), jnp.float32),
            pltpu.VMEM_SHARED((dc * V, 128), jnp.float32),
            pltpu.SemaphoreType.DMA,
            pltpu.SemaphoreType.DMA,
        ],
    )
    def k(*refs):
        tabs = refs[:dc]
        idx_hbm = refs[dc]
        outs = refs[dc + 1:dc + 1 + dc]
        idx_v, rows0, rows1, sh, sem0, sem1 = refs[dc + 1 + dc:]
        w = _wid()
        _stage_tables(tabs, sh, V)
        pltpu.sync_copy(idx_hbm.at[w], idx_v)
        plsc.subcore_barrier()

        def fire(c, rows, sem):
            pltpu.async_copy(sh.at[idx_v.at[c]], rows, sem, add=True)

        def wb(c, rows, sem):
            pltpu.make_async_copy(outs[0].at[pl.ds(0, R)], rows,
                                  sem).wait()
            for cc in range(dc):
                pltpu.sync_copy(
                    rows.at[pl.ds(cc * CB, CB)],
                    outs[cc].at[pl.ds((w * nch + c) * CB, CB)])

        _zero_rows(rows0, R)
        fire(0, rows0, sem0)
        _zero_rows(rows1, R)

        def body(c, carry):
            @pl.when(c % 2 == 1)
            def _():
                fire(c, rows1, sem1)
                wb(c - 1, rows0, sem0)
                _zero_rows(rows0, R)

            @pl.when(c % 2 == 0)
            def _():
                fire(c, rows0, sem0)
                wb(c - 1, rows1, sem1)
                _zero_rows(rows1, R)

            return carry

        lax.fori_loop(1, nch, body, 0)
        if nch % 2 == 1:
            wb(nch - 1, rows0, sem0)
        else:
            wb(nch - 1, rows1, sem1)

    return k(*tables, idx2)


def _tc_mm(xs, Ws, scale=None, relu=True, br=512):
    dci = len(xs)
    N = xs[0].shape[0]
    Do = Ws[0].shape[1]
    dco = Do // 128

    def body(*refs):
        xrefs = refs[:dci]
        wrefs = refs[dci:2 * dci]
        orefs = refs[2 * dci:]
        y = None
        for cc in range(dci):
            xb = xrefs[cc][...]
            if scale is not None:
                xb = xb * scale
            t = jnp.dot(xb, wrefs[cc][...], preferred_element_type=jnp.float32)
            y = t if y is None else y + t
        if relu:
            y = jnp.maximum(y, 0.0)
        for co in range(dco):
            orefs[co][...] = y[:, co * 128:(co + 1) * 128]

    return pl.pallas_call(
        body,
        grid=(N // br,),
        in_specs=(
            [pl.BlockSpec((br, 128), lambda i: (i, 0)) for _ in range(dci)]
            + [pl.BlockSpec((128, Do), lambda i: (0, 0)) for _ in range(dci)]
        ),
        out_specs=[pl.BlockSpec((br, 128), lambda i: (i, 0))
                   for _ in range(dco)],
        out_shape=tuple(
            jax.ShapeDtypeStruct((N, 128), jnp.float32) for _ in range(dco)),
    )(*xs, *Ws)


def _tc_enc2_lat(s2s, W2s, Wl1b):
    dci = len(s2s)
    N = s2s[0].shape[0]

    def body(*refs):
        xrefs = refs[:dci]
        wrefs = refs[dci:2 * dci]
        wb_ref = refs[2 * dci]
        o0, o1 = refs[2 * dci + 1:]
        y = None
        for cc in range(dci):
            t = jnp.dot(xrefs[cc][...] * (1.0 / _K), wrefs[cc][...],
                        preferred_element_type=jnp.float32)
            y = t if y is None else y + t
        x2 = jnp.maximum(y, 0.0)
        z2 = jnp.dot(x2, wb_ref[...], preferred_element_type=jnp.float32)
        o0[...] = z2[:, :128]
        o1[...] = z2[:, 128:]

    return pl.pallas_call(
        body,
        out_shape=(
            jax.ShapeDtypeStruct((N, 128), jnp.float32),
            jax.ShapeDtypeStruct((N, 128), jnp.float32),
        ),
    )(*s2s, *W2s, Wl1b)


def _tc_lat1(x1s, u1s, Wl1as, Wlb, br=512):
    N = x1s[0].shape[0]

    def body(x0_ref, x1_ref, u0_ref, u1_ref, wa0_ref, wa1_ref, wb_ref, o_ref):
        y = (jnp.dot(x0_ref[...], wa0_ref[...],
                     preferred_element_type=jnp.float32)
             + jnp.dot(x1_ref[...], wa1_ref[...],
                       preferred_element_type=jnp.float32))
        u = jnp.concatenate([u0_ref[...], u1_ref[...]], axis=1)
        h = jnp.maximum(y + u, 0.0)
        o_ref[...] = jnp.dot(h, wb_ref[...], preferred_element_type=jnp.float32)

    return pl.pallas_call(
        body,
        grid=(N // br,),
        in_specs=[
            pl.BlockSpec((br, 128), lambda i: (i, 0)),
            pl.BlockSpec((br, 128), lambda i: (i, 0)),
            pl.BlockSpec((br, 128), lambda i: (i, 0)),
            pl.BlockSpec((br, 128), lambda i: (i, 0)),
            pl.BlockSpec((128, 256), lambda i: (0, 0)),
            pl.BlockSpec((128, 256), lambda i: (0, 0)),
            pl.BlockSpec((256, 128), lambda i: (0, 0)),
        ],
        out_specs=pl.BlockSpec((br, 128), lambda i: (i, 0)),
        out_shape=jax.ShapeDtypeStruct((N, 128), jnp.float32),
    )(x1s[0], x1s[1], u1s[0], u1s[1], Wl1as[0], Wl1as[1], Wlb)


def _tc_head(x0, u0, Wla, W_head, W_out, br=1024):
    N, D = x0.shape
    C = W_out.shape[1]

    def body(x_ref, u_ref, wa_ref, wh_ref, wo_ref, o_ref):
        t = jnp.maximum(
            jnp.dot(x_ref[...], wa_ref[...], preferred_element_type=jnp.float32)
            + u_ref[...], 0.0)
        t = jnp.maximum(
            jnp.dot(t, wh_ref[...], preferred_element_type=jnp.float32), 0.0)
        o_ref[...] = jnp.dot(t, wo_ref[...], preferred_element_type=jnp.float32)

    return pl.pallas_call(
        body,
        grid=(N // br,),
        in_specs=[
            pl.BlockSpec((br, D), lambda i: (i, 0)),
            pl.BlockSpec((br, D), lambda i: (i, 0)),
            pl.BlockSpec((D, D), lambda i: (0, 0)),
            pl.BlockSpec((D, D), lambda i: (0, 0)),
            pl.BlockSpec((D, C), lambda i: (0, 0)),
        ],
        out_specs=pl.BlockSpec((br, C), lambda i: (i, 0)),
        out_shape=jax.ShapeDtypeStruct((N, C), jnp.float32),
    )(x0, u0, Wla, W_head, W_out)


def _pack_neigh(n, P, CB, dc, V):
    G = P // CB
    n = jnp.pad(n, ((0, P - n.shape[0]), (0, 0))).astype(jnp.int32)
    q = n.reshape(G, CB, _K)
    off = (jnp.arange(dc, dtype=jnp.int32) * V)
    q = q[:, None, :, :] + off[None, :, None, None]
    q = q.transpose(0, 3, 1, 2).reshape(G, _K, dc * CB)
    return q.reshape(_NW, G // _NW, _K, dc * CB)


def _pack_pool(p, P, CB, dc, V):
    G = P // CB
    p = jnp.pad(p, ((0, P - p.shape[0]), (0, 0))).astype(jnp.int32)
    q = p.reshape(G, 1, CB * _K)
    off = (jnp.arange(dc, dtype=jnp.int32) * V)
    q = q + off[None, :, None]
    return q.reshape(_NW, G // _NW, dc * CB * _K)


def _pack_ups(u, P, CB, dc, V):
    G = P // CB
    u = jnp.pad(u, (0, P - u.shape[0])).astype(jnp.int32)
    q = u.reshape(G, 1, CB)
    off = (jnp.arange(dc, dtype=jnp.int32) * V)
    q = q + off[None, :, None]
    return q.reshape(_NW, G // _NW, dc * CB)


def kernel(features, neighbors0, neighbors1, neighbors2, pools1, pools2,
           upsamples0, upsamples1,
           W_enc0, W_pool1, W_enc1, W_pool2, W_enc2, W_lat1, W_last, W_head,
           W_out):
    N0 = features.shape[0]

    n0 = _pack_neigh(neighbors0, _P0, 32, 1, 0)
    n1 = _pack_neigh(neighbors1, _P1, 16, 2, _P1)
    n2 = _pack_neigh(neighbors2, _P2, 8, 4, _P2)
    p1 = _pack_pool(pools1, _P1, 4, 1, 0)
    p2 = _pack_pool(pools2, _P2, 2, 2, _P1)
    u0 = _pack_ups(upsamples0, _P0, 64, 1, 0)
    u1 = _pack_ups(upsamples1, _P1, 40, 2, _P2)

    W_enc1s = [W_enc1[cc * 128:(cc + 1) * 128] for cc in range(2)]
    W_pool2s = [W_pool2[cc * 128:(cc + 1) * 128] for cc in range(2)]
    W_enc2s = [W_enc2[cc * 128:(cc + 1) * 128] for cc in range(4)]
    Wl1as = [W_lat1[cc * 128:(cc + 1) * 128] for cc in range(2)]
    Wl1b = W_lat1[256:]
    Wla, Wlb = W_last[:128], W_last[128:]

    (s0,) = _sc_gather_sum([features], n0, _P0, 32)
    (x0,) = _tc_mm([s0], [W_enc0], scale=1.0 / _K, br=1024)
    (m1,) = _sc_gather_max([x0], p1, _P1, 4)
    h1 = _tc_mm([m1], [W_pool1], br=512)
    s1 = _sc_gather_sum(list(h1), n1, _P1, 16)
    x1 = _tc_mm(list(s1), W_enc1s, scale=1.0 / _K, br=512)
    m2 = _sc_gather_max(list(x1), p2, _P2, 2)
    h2 = _tc_mm(list(m2), W_pool2s, br=768)
    s2 = _sc_gather_sum(list(h2), n2, _P2, 8)
    z2 = _tc_enc2_lat(list(s2), W_enc2s, Wl1b)

    uu1 = _sc_gather_rows(list(z2), u1, _P1, 40)
    z1 = _tc_lat1(list(x1), list(uu1), Wl1as, Wlb, br=512)
    (uu0,) = _sc_gather_rows([z1], u0, _P0, 64)
    logits = _tc_head(x0, uu0, Wla, W_head, W_out, br=1024)

    return logits[:N0]


# speedup vs baseline: 4.5500x; 1.0027x over previous
import functools

import jax
import jax.numpy as jnp
from jax import lax
from jax.experimental import pallas as pl
from jax.experimental.pallas import tpu as pltpu
import jax.experimental.pallas.tpu_sc as plsc

_K = 32
_NC, _NS = 2, 16
_NW = _NC * _NS
_L = 16

_P0, _P1, _P2 = 10240, 2560, 768


def _mesh():
    return plsc.VectorSubcoreMesh(core_axis_name="c", subcore_axis_name="s",
                                  num_cores=_NC, num_subcores=_NS)


def _wid():
    return lax.axis_index("s") * _NC + lax.axis_index("c")


def _stage_tables(tables, sh, V):
    @pl.when(lax.axis_index("s") == 0)
    def _():
        for cc, t in enumerate(tables):
            pltpu.sync_copy(t, sh.at[pl.ds(cc * V, V)])


def _zero_rows(buf, n):
    z = jnp.zeros((_L,), jnp.float32)

    def zrow(i, carry):
        for dd in range(128 // _L):
            buf[i, pl.ds(dd * _L, _L)] = z
        return carry

    lax.fori_loop(0, n, zrow, 0)


def _sc_gather_sum(tables, idx3, P, CB):
    dc = len(tables)
    R = CB * dc
    G = P // CB
    nch = G // _NW
    V = tables[0].shape[0]

    @functools.partial(
        pl.kernel,
        out_type=tuple(
            jax.ShapeDtypeStruct((P, 128), jnp.float32) for _ in range(dc)),
        mesh=_mesh(),
        scratch_types=[
            pltpu.VMEM((nch, _K, R), jnp.int32),
            pltpu.VMEM((R, 128), jnp.float32),
            pltpu.VMEM((R, 128), jnp.float32),
            pltpu.VMEM_SHARED((dc * V, 128), jnp.float32),
            pltpu.SemaphoreType.DMA,
            pltpu.SemaphoreType.DMA,
        ],
    )
    def k(*refs):
        tabs = refs[:dc]
        idx_hbm = refs[dc]
        outs = refs[dc + 1:dc + 1 + dc]
        idx_v, acc0, acc1, sh, sem0, sem1 = refs[dc + 1 + dc:]
        w = _wid()
        _stage_tables(tabs, sh, V)
        pltpu.sync_copy(idx_hbm.at[w], idx_v)
        plsc.subcore_barrier()

        def fire(c, acc, sem):
            for kk in range(_K):
                pltpu.async_copy(sh.at[idx_v.at[c, kk]], acc, sem, add=True)

        def drain_wb(c, acc, sem):
            for kk in range(_K):
                pltpu.make_async_copy(outs[0].at[pl.ds(0, R)], acc,
                                      sem).wait()
            for cc in range(dc):
                pltpu.sync_copy(
                    acc.at[pl.ds(cc * CB, CB)],
                    outs[cc].at[pl.ds((w * nch + c) * CB, CB)])

        _zero_rows(acc0, R)
        fire(0, acc0, sem0)

        def body(c, carry):
            @pl.when(c % 2 == 1)
            def _():
                _zero_rows(acc1, R)
                fire(c, acc1, sem1)
                drain_wb(c - 1, acc0, sem0)

            @pl.when(c % 2 == 0)
            def _():
                _zero_rows(acc0, R)
                fire(c, acc0, sem0)
                drain_wb(c - 1, acc1, sem1)

            return carry

        lax.fori_loop(1, nch, body, 0)
        if nch % 2 == 1:
            drain_wb(nch - 1, acc0, sem0)
        else:
            drain_wb(nch - 1, acc1, sem1)

    return k(*tables, idx3)


def _sc_gather_max(tables, idx2, P, CB):
    dc = len(tables)
    G = P // CB
    nch = G // _NW
    M = CB * _K
    R = M * dc
    V = tables[0].shape[0]

    @functools.partial(
        pl.kernel,
        out_type=tuple(
            jax.ShapeDtypeStruct((P, 128), jnp.float32) for _ in range(dc)),
        mesh=_mesh(),
        scratch_types=[
            pltpu.VMEM((nch, R), jnp.int32),
            pltpu.VMEM((R, 128), jnp.float32),
            pltpu.VMEM((R, 128), jnp.float32),
            pltpu.VMEM((dc, CB, 128), jnp.float32),
            pltpu.VMEM_SHARED((dc * V, 128), jnp.float32),
            pltpu.SemaphoreType.DMA,
            pltpu.SemaphoreType.DMA,
        ],
    )
    def k(*refs):
        tabs = refs[:dc]
        idx_hbm = refs[dc]
        outs = refs[dc + 1:dc + 1 + dc]
        idx_v, rows0, rows1, out_v, sh, sem0, sem1 = refs[dc + 1 + dc:]
        w = _wid()
        _stage_tables(tabs, sh, V)
        pltpu.sync_copy(idx_hbm.at[w], idx_v)
        plsc.subcore_barrier()

        def fire(c, rows, sem):
            pltpu.async_copy(sh.at[idx_v.at[c]], rows, sem, add=True)

        def compute_wb(c, rows_v):
            for p in range(CB):
                for cc in range(dc):
                    base = (cc * CB + p) * _K
                    init = tuple(
                        rows_v[base, pl.ds(dd * _L, _L)]
                        for dd in range(128 // _L)
                    )

                    def kbody(kk, m):
                        return tuple(
                            jnp.maximum(m[dd],
                                        rows_v[base + kk, pl.ds(dd * _L, _L)])
                            for dd in range(128 // _L)
                        )

                    m = lax.fori_loop(1, _K, kbody, init)
                    for dd in range(128 // _L):
                        out_v[cc, p, pl.ds(dd * _L, _L)] = m[dd]
            for cc in range(dc):
                pltpu.sync_copy(
                    out_v.at[cc],
                    outs[cc].at[pl.ds((w * nch + c) * CB, CB)])

        _zero_rows(rows0, R)
        fire(0, rows0, sem0)
        _zero_rows(rows1, R)

        def body(c, carry):
            @pl.when(c % 2 == 1)
            def _():
                fire(c, rows1, sem1)
                pltpu.make_async_copy(outs[0].at[pl.ds(0, R)], rows0,
                                      sem0).wait()
                compute_wb(c - 1, rows0)
                _zero_rows(rows0, R)

            @pl.when(c % 2 == 0)
            def _():
                fire(c, rows0, sem0)
                pltpu.make_async_copy(outs[0].at[pl.ds(0, R)], rows1,
                                      sem1).wait()
                compute_wb(c - 1, rows1)
                _zero_rows(rows1, R)

            return carry

        lax.fori_loop(1, nch, body, 0)
        if nch % 2 == 1:
            pltpu.make_async_copy(outs[0].at[pl.ds(0, R)], rows0,
                                      sem0).wait()
            compute_wb(nch - 1, rows0)
        else:
            pltpu.make_async_copy(outs[0].at[pl.ds(0, R)], rows1,
                                      sem1).wait()
            compute_wb(nch - 1, rows1)

    return k(*tables, idx2)


def _sc_gather_rows(tables, idx2, P, CB):
    dc = len(tables)
    R = CB * dc
    G = P // CB
    nch = G // _NW
    V = tables[0].shape[0]

    @functools.partial(
        pl.kernel,
        out_type=tuple(
            jax.ShapeDtypeStruct((P, 128), jnp.float32) for _ in range(dc)),
        mesh=_mesh(),
        scratch_types=[
            pltpu.VMEM((nch, R), jnp.int32),
            pltpu.VMEM((R, 128), jnp.float32),
            pltpu.VMEM((R, 128), jnp.float32),
            pltpu.VMEM_SHARED((dc * V, 128), jnp.float32),
            pltpu.SemaphoreType.DMA,
            pltpu.SemaphoreType.DMA,
        ],
    )
    def k(*refs):
        tabs = refs[:dc]
        idx_hbm = refs[dc]
        outs = refs[dc + 1:dc + 1 + dc]
        idx_v, rows0, rows1, sh, sem0, sem1 = refs[dc + 1 + dc:]
        w = _wid()
        _stage_tables(tabs, sh, V)
        pltpu.sync_copy(idx_hbm.at[w], idx_v)
        plsc.subcore_barrier()

        def fire(c, rows, sem):
            pltpu.async_copy(sh.at[idx_v.at[c]], rows, sem, add=True)

        def wb(c, rows, sem):
            pltpu.make_async_copy(outs[0].at[pl.ds(0, R)], rows,
                                  sem).wait()
            for cc in range(dc):
                pltpu.sync_copy(
                    rows.at[pl.ds(cc * CB, CB)],
                    outs[cc].at[pl.ds((w * nch + c) * CB, CB)])

        _zero_rows(rows0, R)
        fire(0, rows0, sem0)
        _zero_rows(rows1, R)

        def body(c, carry):
            @pl.when(c % 2 == 1)
            def _():
                fire(c, rows1, sem1)
                wb(c - 1, rows0, sem0)
                _zero_rows(rows0, R)

            @pl.when(c % 2 == 0)
            def _():
                fire(c, rows0, sem0)
                wb(c - 1, rows1, sem1)
                _zero_rows(rows1, R)

            return carry

        lax.fori_loop(1, nch, body, 0)
        if nch % 2 == 1:
            wb(nch - 1, rows0, sem0)
        else:
            wb(nch - 1, rows1, sem1)

    return k(*tables, idx2)


def _tc_mm(xs, Ws, scale=None, relu=True, br=512):
    dci = len(xs)
    N = xs[0].shape[0]
    Do = Ws[0].shape[1]
    dco = Do // 128

    def body(*refs):
        xrefs = refs[:dci]
        wrefs = refs[dci:2 * dci]
        orefs = refs[2 * dci:]
        y = None
        for cc in range(dci):
            xb = xrefs[cc][...]
            if scale is not None:
                xb = xb * scale
            t = jnp.dot(xb, wrefs[cc][...], preferred_element_type=jnp.float32)
            y = t if y is None else y + t
        if relu:
            y = jnp.maximum(y, 0.0)
        for co in range(dco):
            orefs[co][...] = y[:, co * 128:(co + 1) * 128]

    return pl.pallas_call(
        body,
        grid=(N // br,),
        in_specs=(
            [pl.BlockSpec((br, 128), lambda i: (i, 0)) for _ in range(dci)]
            + [pl.BlockSpec((128, Do), lambda i: (0, 0)) for _ in range(dci)]
        ),
        out_specs=[pl.BlockSpec((br, 128), lambda i: (i, 0))
                   for _ in range(dco)],
        out_shape=tuple(
            jax.ShapeDtypeStruct((N, 128), jnp.float32) for _ in range(dco)),
    )(*xs, *Ws)


def _tc_enc2_lat(s2s, W2s, Wl1b):
    dci = len(s2s)
    N = s2s[0].shape[0]

    def body(*refs):
        xrefs = refs[:dci]
        wrefs = refs[dci:2 * dci]
        wb_ref = refs[2 * dci]
        o0, o1 = refs[2 * dci + 1:]
        y = None
        for cc in range(dci):
            t = jnp.dot(xrefs[cc][...] * (1.0 / _K), wrefs[cc][...],
                        preferred_element_type=jnp.float32)
            y = t if y is None else y + t
        x2 = jnp.maximum(y, 0.0)
        z2 = jnp.dot(x2, wb_ref[...], preferred_element_type=jnp.float32)
        o0[...] = z2[:, :128]
        o1[...] = z2[:, 128:]

    return pl.pallas_call(
        body,
        out_shape=(
            jax.ShapeDtypeStruct((N, 128), jnp.float32),
            jax.ShapeDtypeStruct((N, 128), jnp.float32),
        ),
    )(*s2s, *W2s, Wl1b)


def _tc_lat1(x1s, u1s, Wl1as, Wlb, br=512):
    N = x1s[0].shape[0]

    def body(x0_ref, x1_ref, u0_ref, u1_ref, wa0_ref, wa1_ref, wb_ref, o_ref):
        y = (jnp.dot(x0_ref[...], wa0_ref[...],
                     preferred_element_type=jnp.float32)
             + jnp.dot(x1_ref[...], wa1_ref[...],
                       preferred_element_type=jnp.float32))
        u = jnp.concatenate([u0_ref[...], u1_ref[...]], axis=1)
        h = jnp.maximum(y + u, 0.0)
        o_ref[...] = jnp.dot(h, wb_ref[...], preferred_element_type=jnp.float32)

    return pl.pallas_call(
        body,
        grid=(N // br,),
        in_specs=[
            pl.BlockSpec((br, 128), lambda i: (i, 0)),
            pl.BlockSpec((br, 128), lambda i: (i, 0)),
            pl.BlockSpec((br, 128), lambda i: (i, 0)),
            pl.BlockSpec((br, 128), lambda i: (i, 0)),
            pl.BlockSpec((128, 256), lambda i: (0, 0)),
            pl.BlockSpec((128, 256), lambda i: (0, 0)),
            pl.BlockSpec((256, 128), lambda i: (0, 0)),
        ],
        out_specs=pl.BlockSpec((br, 128), lambda i: (i, 0)),
        out_shape=jax.ShapeDtypeStruct((N, 128), jnp.float32),
    )(x1s[0], x1s[1], u1s[0], u1s[1], Wl1as[0], Wl1as[1], Wlb)


def _tc_head(x0, u0, Wla, W_head, W_out, N, br=1000):
    D = x0.shape[1]
    C = W_out.shape[1]

    def body(x_ref, u_ref, wa_ref, wh_ref, wo_ref, o_ref):
        t = jnp.maximum(
            jnp.dot(x_ref[...], wa_ref[...], preferred_element_type=jnp.float32)
            + u_ref[...], 0.0)
        t = jnp.maximum(
            jnp.dot(t, wh_ref[...], preferred_element_type=jnp.float32), 0.0)
        o_ref[...] = jnp.dot(t, wo_ref[...], preferred_element_type=jnp.float32)

    return pl.pallas_call(
        body,
        grid=(N // br,),
        in_specs=[
            pl.BlockSpec((br, D), lambda i: (i, 0)),
            pl.BlockSpec((br, D), lambda i: (i, 0)),
            pl.BlockSpec((D, D), lambda i: (0, 0)),
            pl.BlockSpec((D, D), lambda i: (0, 0)),
            pl.BlockSpec((D, C), lambda i: (0, 0)),
        ],
        out_specs=pl.BlockSpec((br, C), lambda i: (i, 0)),
        out_shape=jax.ShapeDtypeStruct((N, C), jnp.float32),
    )(x0, u0, Wla, W_head, W_out)


def _pack_neigh(n, P, CB, dc, V):
    G = P // CB
    n = jnp.pad(n, ((0, P - n.shape[0]), (0, 0))).astype(jnp.int32)
    q = n.reshape(G, CB, _K)
    off = (jnp.arange(dc, dtype=jnp.int32) * V)
    q = q[:, None, :, :] + off[None, :, None, None]
    q = q.transpose(0, 3, 1, 2).reshape(G, _K, dc * CB)
    return q.reshape(_NW, G // _NW, _K, dc * CB)


def _pack_pool(p, P, CB, dc, V):
    G = P // CB
    p = jnp.pad(p, ((0, P - p.shape[0]), (0, 0))).astype(jnp.int32)
    q = p.reshape(G, 1, CB * _K)
    off = (jnp.arange(dc, dtype=jnp.int32) * V)
    q = q + off[None, :, None]
    return q.reshape(_NW, G // _NW, dc * CB * _K)


def _pack_ups(u, P, CB, dc, V):
    G = P // CB
    u = jnp.pad(u, (0, P - u.shape[0])).astype(jnp.int32)
    q = u.reshape(G, 1, CB)
    off = (jnp.arange(dc, dtype=jnp.int32) * V)
    q = q + off[None, :, None]
    return q.reshape(_NW, G // _NW, dc * CB)


def kernel(features, neighbors0, neighbors1, neighbors2, pools1, pools2,
           upsamples0, upsamples1,
           W_enc0, W_pool1, W_enc1, W_pool2, W_enc2, W_lat1, W_last, W_head,
           W_out):
    N0 = features.shape[0]

    n0 = _pack_neigh(neighbors0, _P0, 32, 1, 0)
    n1 = _pack_neigh(neighbors1, _P1, 16, 2, _P1)
    n2 = _pack_neigh(neighbors2, _P2, 8, 4, _P2)
    p1 = _pack_pool(pools1, _P1, 4, 1, 0)
    p2 = _pack_pool(pools2, _P2, 2, 2, _P1)
    u0 = _pack_ups(upsamples0, _P0, 64, 1, 0)
    u1 = _pack_ups(upsamples1, _P1, 40, 2, _P2)

    W_enc1s = [W_enc1[cc * 128:(cc + 1) * 128] for cc in range(2)]
    W_pool2s = [W_pool2[cc * 128:(cc + 1) * 128] for cc in range(2)]
    W_enc2s = [W_enc2[cc * 128:(cc + 1) * 128] for cc in range(4)]
    Wl1as = [W_lat1[cc * 128:(cc + 1) * 128] for cc in range(2)]
    Wl1b = W_lat1[256:]
    Wla, Wlb = W_last[:128], W_last[128:]

    (s0,) = _sc_gather_sum([features], n0, _P0, 32)
    (x0,) = _tc_mm([s0], [W_enc0], scale=1.0 / _K, br=1024)
    (m1,) = _sc_gather_max([x0], p1, _P1, 4)
    h1 = _tc_mm([m1], [W_pool1], br=512)
    s1 = _sc_gather_sum(list(h1), n1, _P1, 16)
    x1 = _tc_mm(list(s1), W_enc1s, scale=1.0 / _K, br=512)
    m2 = _sc_gather_max(list(x1), p2, _P2, 2)
    h2 = _tc_mm(list(m2), W_pool2s, br=768)
    s2 = _sc_gather_sum(list(h2), n2, _P2, 8)
    z2 = _tc_enc2_lat(list(s2), W_enc2s, Wl1b)

    uu1 = _sc_gather_rows(list(z2), u1, _P1, 40)
    z1 = _tc_lat1(list(x1), list(uu1), Wl1as, Wlb, br=512)
    (uu0,) = _sc_gather_rows([z1], u0, _P0, 64)
    return _tc_head(x0, uu0, Wla, W_head, W_out, N0)
